# baseline (speedup 1.0000x reference)
import jax
import jax.numpy as jnp
from jax.experimental import pallas as pl
from jax.experimental.pallas import tpu as pltpu

_LG = 128


def _body(q_ref, ones_ref, m_ref, o_ref, best_s_ref, best_row_ref):
    i = pl.program_id(0)

    @pl.when(i == 0)
    def _init():
        best_s_ref[0] = -jnp.inf

    block = m_ref[...]
    d = block.shape[1]
    nchunk = d // _LG
    pd = block[:, 0:_LG] * q_ref[0]
    pn = block[:, 0:_LG] * block[:, 0:_LG]
    for c in range(1, nchunk):
        col = block[:, c * _LG:(c + 1) * _LG]
        pd = pd + col * q_ref[c]
        pn = pn + col * col
    ones = ones_ref[...]
    dots = jnp.dot(pd, ones, preferred_element_type=jnp.float32)
    nrm = jnp.dot(pn, ones, preferred_element_type=jnp.float32)
    score = dots * jnp.abs(dots) / jnp.maximum(nrm, 1e-30)
    bmax = jnp.max(score)

    @pl.when(bmax > best_s_ref[0])
    def _upd():
        idx = jnp.argmax(score[:, 0])
        best_s_ref[0] = bmax
        best_row_ref[...] = m_ref[pl.ds(idx, 1), :]

    @pl.when(i == pl.num_programs(0) - 1)
    def _fin():
        o_ref[...] = best_row_ref[0, :]


@jax.jit
def kernel(query, memory):
    k, d = memory.shape
    b = 5000
    assert k % b == 0 and d % _LG == 0
    grid = k // b
    q2 = query.reshape(d // _LG, _LG)
    ones = jnp.ones((_LG, 1), jnp.float32)
    out = pl.pallas_call(
        _body,
        grid=(grid,),
        in_specs=[
            pl.BlockSpec((d // _LG, _LG), lambda i: (0, 0)),
            pl.BlockSpec((_LG, 1), lambda i: (0, 0)),
            pl.BlockSpec((b, d), lambda i: (i, 0)),
        ],
        out_specs=pl.BlockSpec((d,), lambda i: (0,)),
        out_shape=jax.ShapeDtypeStruct((d,), jnp.float32),
        scratch_shapes=[
            pltpu.SMEM((1,), jnp.float32),
            pltpu.VMEM((1, d), jnp.float32),
        ],
    )(q2, ones, memory)
    return out


# speedup vs baseline: 1.0213x; 1.0213x over previous
import jax
import jax.numpy as jnp
from jax.experimental import pallas as pl
from jax.experimental.pallas import tpu as pltpu

_LG = 128


def _body(q_ref, ones_ref, m_ref, o_ref, best_s_ref, best_row_ref):
    i = pl.program_id(0)

    @pl.when(i == 0)
    def _init():
        best_s_ref[0] = -jnp.inf

    block = m_ref[...]
    d = block.shape[1]
    nchunk = d // _LG
    pd = block[:, 0:_LG] * q_ref[0]
    pn = block[:, 0:_LG] * block[:, 0:_LG]
    for c in range(1, nchunk):
        col = block[:, c * _LG:(c + 1) * _LG]
        pd = pd + col * q_ref[c]
        pn = pn + col * col
    ones = ones_ref[...]
    cdims = (((1,), (1,)), ((), ()))
    dots = jax.lax.dot_general(ones, pd, cdims,
                               preferred_element_type=jnp.float32)
    nrm = jax.lax.dot_general(ones, pn, cdims,
                              preferred_element_type=jnp.float32)
    score = dots * jnp.abs(dots) / jnp.maximum(nrm, 1e-30)
    bmax = jnp.max(score)

    @pl.when(bmax > best_s_ref[0])
    def _upd():
        idx = jnp.argmax(score[0, :])
        best_s_ref[0] = bmax
        best_row_ref[...] = m_ref[pl.ds(idx, 1), :]

    @pl.when(i == pl.num_programs(0) - 1)
    def _fin():
        o_ref[...] = best_row_ref[0, :]


@jax.jit
def kernel(query, memory):
    k, d = memory.shape
    b = 4000
    assert k % b == 0 and d % _LG == 0
    grid = k // b
    q2 = query.reshape(d // _LG, _LG)
    ones = jnp.ones((1, _LG), jnp.float32)
    out = pl.pallas_call(
        _body,
        grid=(grid,),
        in_specs=[
            pl.BlockSpec((d // _LG, _LG), lambda i: (0, 0)),
            pl.BlockSpec((1, _LG), lambda i: (0, 0)),
            pl.BlockSpec((b, d), lambda i: (i, 0)),
        ],
        out_specs=pl.BlockSpec((d,), lambda i: (0,)),
        out_shape=jax.ShapeDtypeStruct((d,), jnp.float32),
        scratch_shapes=[
            pltpu.SMEM((1,), jnp.float32),
            pltpu.VMEM((1, d), jnp.float32),
        ],
    )(q2, ones, memory)
    return out
